# baseline (speedup 1.0000x reference)
import functools

import jax
import jax.numpy as jnp
from jax import lax
from jax.experimental import pallas as pl
from jax.experimental.pallas import tpu as pltpu
from jax.experimental.pallas import tpu_sc as plsc

_VOCAB_ROWS = 100002
_EMB_DIM = 16
_BATCH = 16384

_info = plsc.get_sparse_core_info()
_NC, _NS = _info.num_cores, _info.num_subcores
_NW = _NC * _NS
_B_PER_W = _BATCH // _NW

_mesh = plsc.VectorSubcoreMesh(core_axis_name="c", subcore_axis_name="s")


@functools.partial(
    pl.kernel,
    mesh=_mesh,
    out_type=jax.ShapeDtypeStruct((_BATCH, _EMB_DIM), jnp.float32),
    compiler_params=pltpu.CompilerParams(use_tc_tiling_on_sc=False),
    scratch_types=[
        pltpu.VMEM((_B_PER_W,), jnp.int32),
        pltpu.VMEM((_B_PER_W, _EMB_DIM), jnp.float32),
        pltpu.SemaphoreType.DMA,
    ],
)
def _sc_gather(table_hbm, idx_hbm, out_hbm, idx_v, rows_v, sem):
    wid = lax.axis_index("s") * _NC + lax.axis_index("c")
    base = wid * _B_PER_W
    pltpu.sync_copy(idx_hbm.at[pl.ds(base, _B_PER_W)], idx_v)
    pltpu.async_copy(table_hbm.at[idx_v], rows_v, sem).wait()
    pltpu.sync_copy(rows_v, out_hbm.at[pl.ds(base, _B_PER_W)])


@jax.jit
def kernel(inputs, table):
    idx = inputs.reshape(-1).astype(jnp.int32)
    return _sc_gather(table, idx)


# speedup vs baseline: 1.1011x; 1.1011x over previous
import functools

import jax
import jax.numpy as jnp
from jax import lax
from jax.experimental import pallas as pl
from jax.experimental.pallas import tpu as pltpu
from jax.experimental.pallas import tpu_sc as plsc

_VOCAB_ROWS = 100002
_EMB_DIM = 16
_BATCH = 16384

_info = plsc.get_sparse_core_info()
_NC, _NS, _L = _info.num_cores, _info.num_subcores, _info.num_lanes
_NW = _NC * _NS
_B_PER_W = _BATCH // _NW
_CB = _B_PER_W // 128

_mesh = plsc.VectorSubcoreMesh(core_axis_name="c", subcore_axis_name="s")


@functools.partial(
    pl.kernel,
    mesh=_mesh,
    out_type=jax.ShapeDtypeStruct((2, _BATCH // 128, 8, 128), jnp.float32),
    compiler_params=pltpu.CompilerParams(
        use_tc_tiling_on_sc=False, needs_layout_passes=False
    ),
    scratch_types=[
        pltpu.VMEM((_B_PER_W,), jnp.int32),
        pltpu.VMEM((_B_PER_W, _EMB_DIM), jnp.float32),
        pltpu.VMEM((2, _CB, 8, 128), jnp.float32),
        pltpu.SemaphoreType.DMA,
    ],
)
def _sc_gather(table_hbm, idx_hbm, out_hbm, idx_v, rows_v, t_v, sem):
    wid = lax.axis_index("s") * _NC + lax.axis_index("c")
    base = wid * _B_PER_W
    pltpu.sync_copy(idx_hbm.at[pl.ds(base, _B_PER_W)], idx_v)
    pltpu.async_copy(table_hbm.at[idx_v], rows_v, sem).wait()
    lane = lax.iota(jnp.int32, _L)
    i0 = lane >> 3
    i2 = lane & 7

    def body(c, carry):
        row = rows_v[c]
        i1 = jnp.full((_L,), c >> 7, jnp.int32)
        i3 = jnp.full((_L,), c & 127, jnp.int32)
        plsc.store_scatter(t_v, [i0, i1, i2, i3], row)
        return carry

    lax.fori_loop(0, _B_PER_W, body, 0, unroll=8)
    pltpu.sync_copy(t_v.at[0], out_hbm.at[0, pl.ds(wid * _CB, _CB)])
    pltpu.sync_copy(t_v.at[1], out_hbm.at[1, pl.ds(wid * _CB, _CB)])


@jax.jit
def kernel(inputs, table):
    idx = inputs.reshape(-1).astype(jnp.int32)
    r = _sc_gather(table, idx)
    return r.transpose(1, 3, 0, 2).reshape(_BATCH, _EMB_DIM)


# speedup vs baseline: 2.0261x; 1.8400x over previous
import functools

import jax
import jax.numpy as jnp
from jax import lax
from jax.experimental import pallas as pl
from jax.experimental.pallas import tpu as pltpu
from jax.experimental.pallas import tpu_sc as plsc

_VOCAB_ROWS = 100002
_EMB_DIM = 16
_BATCH = 16384

_info = plsc.get_sparse_core_info()
_NC, _NS, _L = _info.num_cores, _info.num_subcores, _info.num_lanes
_NW = _NC * _NS
_B_PER_W = _BATCH // _NW
_CB = _B_PER_W // 128

_mesh = plsc.VectorSubcoreMesh(core_axis_name="c", subcore_axis_name="s")


@functools.partial(
    pl.kernel,
    mesh=_mesh,
    out_type=jax.ShapeDtypeStruct((2, _BATCH // 128, 8, 128), jnp.float32),
    compiler_params=pltpu.CompilerParams(
        use_tc_tiling_on_sc=False, needs_layout_passes=False
    ),
    scratch_types=[
        pltpu.VMEM((_B_PER_W,), jnp.int32),
        pltpu.VMEM((_EMB_DIM, _B_PER_W), jnp.int32),
        pltpu.VMEM((_EMB_DIM, _B_PER_W), jnp.float32),
        pltpu.SemaphoreType.DMA,
    ],
)
def _sc_gather(table_hbm, idx_hbm, out_hbm, idx_v, idx2_v, t_v, sem):
    wid = lax.axis_index("s") * _NC + lax.axis_index("c")
    base = wid * _B_PER_W
    pltpu.sync_copy(idx_hbm.at[pl.ds(base, _B_PER_W)], idx_v)

    def body(i, carry):
        v = idx_v[pl.ds(i * _L, _L)]
        for d in range(_EMB_DIM):
            idx2_v[d, pl.ds(i * _L, _L)] = v + (d * _VOCAB_ROWS)
        return carry

    lax.fori_loop(0, _B_PER_W // _L, body, 0, unroll=2)

    descs = [
        pltpu.async_copy(table_hbm.at[idx2_v.at[d]], t_v.at[d], sem)
        for d in range(_EMB_DIM)
    ]
    for dsc in descs:
        dsc.wait()

    for tr in range(2):
        for j in range(_CB):
            pltpu.sync_copy(
                t_v.at[pl.ds(8 * tr, 8), pl.ds(128 * j, 128)],
                out_hbm.at[tr, wid * _CB + j],
            )


@jax.jit
def kernel(inputs, table):
    idx = inputs.reshape(-1).astype(jnp.int32)
    tlin = table.T.reshape(-1)
    r = _sc_gather(tlin, idx)
    return r.transpose(1, 3, 0, 2).reshape(_BATCH, _EMB_DIM)


# speedup vs baseline: 2.0513x; 1.0124x over previous
import functools

import jax
import jax.numpy as jnp
from jax import lax
from jax.experimental import pallas as pl
from jax.experimental.pallas import tpu as pltpu
from jax.experimental.pallas import tpu_sc as plsc

_VOCAB_ROWS = 100002
_EMB_DIM = 16
_BATCH = 16384

_info = plsc.get_sparse_core_info()
_NC, _NS, _L = _info.num_cores, _info.num_subcores, _info.num_lanes
_NW = _NC * _NS
_B_PER_W = _BATCH // _NW
_CB = _B_PER_W // 128

_mesh = plsc.VectorSubcoreMesh(core_axis_name="c", subcore_axis_name="s")


@functools.partial(
    pl.kernel,
    mesh=_mesh,
    out_type=jax.ShapeDtypeStruct((2, _BATCH // 128, 8, 128), jnp.float32),
    compiler_params=pltpu.CompilerParams(
        use_tc_tiling_on_sc=False, needs_layout_passes=False
    ),
    scratch_types=[
        pltpu.VMEM((_B_PER_W,), jnp.int32),
        pltpu.VMEM((_EMB_DIM, _B_PER_W), jnp.int32),
        pltpu.VMEM((_EMB_DIM, _B_PER_W), jnp.float32),
        pltpu.SemaphoreType.DMA,
        pltpu.SemaphoreType.DMA,
    ],
)
def _sc_gather(table_hbm, idx_hbm, out_hbm, idx_v, idx2_v, t_v, sem0, sem1):
    wid = lax.axis_index("s") * _NC + lax.axis_index("c")
    base = wid * _B_PER_W
    pltpu.sync_copy(idx_hbm.at[pl.ds(base, _B_PER_W)], idx_v)

    n_chunks = _B_PER_W // _L

    def make_idx_row(d):
        def body(i, carry):
            v = idx_v[pl.ds(i * _L, _L)]
            idx2_v[d, pl.ds(i * _L, _L)] = v + (d * _VOCAB_ROWS)
            return carry

        lax.fori_loop(0, n_chunks, body, 0, unroll=4)

    descs = []
    for d in range(_EMB_DIM):
        make_idx_row(d)
        sem = sem0 if d < 8 else sem1
        descs.append(pltpu.async_copy(table_hbm.at[idx2_v.at[d]], t_v.at[d], sem))

    for d in range(8):
        descs[d].wait()
    for j in range(_CB):
        pltpu.sync_copy(
            t_v.at[pl.ds(0, 8), pl.ds(128 * j, 128)],
            out_hbm.at[0, wid * _CB + j],
        )
    for d in range(8, _EMB_DIM):
        descs[d].wait()
    for j in range(_CB):
        pltpu.sync_copy(
            t_v.at[pl.ds(8, 8), pl.ds(128 * j, 128)],
            out_hbm.at[1, wid * _CB + j],
        )


@jax.jit
def kernel(inputs, table):
    idx = inputs.reshape(-1).astype(jnp.int32)
    tlin = table.T.reshape(-1)
    r = _sc_gather(tlin, idx)
    return r.transpose(1, 3, 0, 2).reshape(_BATCH, _EMB_DIM)
